# Initial kernel scaffold; baseline (speedup 1.0000x reference)
import jax
import jax.numpy as jnp
from jax.experimental import pallas as pl


def kernel(features, edge_index0, pool_idx1, edge_index1, pool_idx2, edge_index2, pool_idx3, edge_index3, W_e0, W_e1, W_e2, W_l1a, W_l1b, W_l1c, W_l2a, W_l2b, W_l2c, W_x1, W_x2, W_out, b_out):
    raise NotImplementedError("write your pallas kernel here")


# speedup vs baseline: 3.6119x; 3.6119x over previous
import functools

import jax
import jax.numpy as jnp
from jax import lax
from jax.experimental import pallas as pl
from jax.experimental.pallas import tpu as pltpu
from jax.experimental.pallas import tpu_sc as plsc

_N0, _N1, _N2, _N3 = 50000, 12500, 3125, 800
_LATENT = 256
_NW = 32
_SPMEM_WORDS = 2097151 - 24576


def _rup(x, m):
    return -(-x // m) * m


def _chunk_width(n_pad, d):
    for dc in (1024, 512, 256, 128, 64, 32, 16):
        if dc <= d and d % dc == 0 and n_pad * dc <= _SPMEM_WORDS - 16 * 4096:
            return dc
    raise ValueError((n_pad, d))


def _block_params(n_pad, dc):
    tile_bytes = min(300 * 1024, (_SPMEM_WORDS - n_pad * dc) * 4 // 16)
    sub = 128
    for sub in (128, 64, 32, 16):
        if sub * dc * 4 + sub * 8 <= tile_bytes:
            break
    kb = max(1, min(16, tile_bytes // (sub * dc * 4 + sub * 8)))
    return sub, kb


@functools.lru_cache(maxsize=None)
def _agg_kernel(n_pad, dc, e_rows, kb, sub):
    rows_per_worker = e_rows // _NW
    iters = rows_per_worker // kb
    nbz = n_pad // sub
    nbz_it = -(-nbz // 16)
    mesh = plsc.VectorSubcoreMesh(core_axis_name="c", subcore_axis_name="s")

    @functools.partial(
        pl.kernel,
        out_type=jax.ShapeDtypeStruct((2 * n_pad, dc), jnp.float32),
        mesh=mesh,
        scratch_types=[
            pltpu.VMEM((kb, sub), jnp.int32),
            pltpu.VMEM((kb, sub), jnp.int32),
            pltpu.VMEM((kb, sub, dc), jnp.float32),
            pltpu.VMEM_SHARED((n_pad, dc), jnp.float32),
            pltpu.SemaphoreType.DMA,
            pltpu.SemaphoreType.DMA,
        ],
        compiler_params=pltpu.CompilerParams(use_tc_tiling_on_sc=False),
    )
    def k(src_hbm, dst_hbm, x_hbm, zeros_hbm, out_hbm, src_v, dst_v, rows_v,
          acc, sem, isem):
        cid = lax.axis_index("c")
        sid = lax.axis_index("s")
        wid = cid * 16 + sid

        def zbody(i, carry):
            b = sid + i * 16

            @pl.when(b < nbz)
            def _():
                pltpu.sync_copy(zeros_hbm.at[pl.ds(b * sub, sub)],
                                acc.at[pl.ds(b * sub, sub)])
            return carry

        lax.fori_loop(0, nbz_it, zbody, 0)
        plsc.subcore_barrier()

        wrow = wid * rows_per_worker

        def ebody(t, carry):
            brow = wrow + t * kb
            ci = pltpu.async_copy(src_hbm.at[pl.ds(brow, kb)], src_v, isem)
            ci2 = pltpu.async_copy(dst_hbm.at[pl.ds(brow, kb)], dst_v, isem)
            ci.wait()
            ci2.wait()
            cps = [pltpu.async_copy(x_hbm.at[src_v.at[j]], rows_v.at[j], sem)
                   for j in range(kb)]
            for cp in cps:
                cp.wait()
            for j in range(kb):
                pltpu.sync_copy(rows_v.at[j], acc.at[dst_v.at[j]], add=True)
            return carry

        lax.fori_loop(0, iters, ebody, 0)
        plsc.subcore_barrier()

        def dbody(i, carry):
            b = sid + i * 16

            @pl.when(b < nbz)
            def _():
                pltpu.sync_copy(acc.at[pl.ds(b * sub, sub)],
                                out_hbm.at[pl.ds(cid * n_pad + b * sub, sub)])
            return carry

        lax.fori_loop(0, nbz_it, dbody, 0)

    return k


def _aggregate(x, src, dst, n_pad):
    d = x.shape[1]
    dc = _chunk_width(n_pad, d)
    sub, kb = _block_params(n_pad, dc)
    unit = _NW * kb * sub
    e = src.shape[0]
    e_pad = _rup(e, unit)
    srcp = jnp.pad(src, (0, e_pad - e)).reshape(e_pad // sub, sub)
    dstp = jnp.pad(dst, (0, e_pad - e),
                   constant_values=n_pad - 1).reshape(e_pad // sub, sub)
    zeros = jnp.zeros((n_pad, dc), jnp.float32)
    parts = []
    for c0 in range(0, d, dc):
        xc = x[:, c0:c0 + dc]
        out = _agg_kernel(n_pad, dc, e_pad // sub, kb, sub)(srcp, dstp, xc,
                                                            zeros)
        parts.append(out.reshape(2, n_pad, dc))
    return parts[0] if len(parts) == 1 else jnp.concatenate(parts, axis=2)


@functools.lru_cache(maxsize=None)
def _pool_kernel(n_out_pad, d):
    nb = n_out_pad // 128
    nb_it = -(-nb // _NW)
    mesh = plsc.VectorSubcoreMesh(core_axis_name="c", subcore_axis_name="s")

    @functools.partial(
        pl.kernel,
        out_type=jax.ShapeDtypeStruct((n_out_pad, d), jnp.float32),
        mesh=mesh,
        scratch_types=[
            pltpu.VMEM((128,), jnp.int32),
            pltpu.VMEM((128, d), jnp.float32),
            pltpu.SemaphoreType.DMA,
        ],
        compiler_params=pltpu.CompilerParams(use_tc_tiling_on_sc=False),
    )
    def k(idx_hbm, x_hbm, out_hbm, idx_v, rows_v, sem):
        cid = lax.axis_index("c")
        sid = lax.axis_index("s")
        wid = cid * 16 + sid

        def body(i, carry):
            b = wid + i * _NW

            @pl.when(b < nb)
            def _():
                pltpu.sync_copy(idx_hbm.at[b], idx_v)
                pltpu.async_copy(x_hbm.at[idx_v], rows_v, sem).wait()
                pltpu.sync_copy(rows_v, out_hbm.at[pl.ds(b * 128, 128)])
            return carry

        lax.fori_loop(0, nb_it, body, 0)

    return k


def _pool(x, idx, n_out_pad):
    d = x.shape[1]
    n = idx.shape[0]
    idxp = jnp.pad(idx, (0, n_out_pad - n)).reshape(n_out_pad // 128, 128)
    return _pool_kernel(n_out_pad, d)(idxp, x)


def _lrelu(t):
    return jnp.where(t >= 0, t, 0.1 * t)


def _norm(p0, p1, g0, g1):
    deg = jnp.maximum(g0[:, 0:1] + g1[:, 0:1], 1.0)
    return (p0[...] + p1[...]) / deg


def _stage_body(p0, p1, g0, g1, w, o):
    a = _norm(p0, p1, g0, g1)
    o[...] = _lrelu(jnp.dot(a, w[...], preferred_element_type=jnp.float32,
                            precision=lax.Precision.HIGHEST))


def _stage_body_res(p0, p1, g0, g1, w, r, o):
    a = _norm(p0, p1, g0, g1)
    o[...] = _lrelu(r[...] + jnp.dot(a, w[...],
                                     preferred_element_type=jnp.float32,
                                     precision=lax.Precision.HIGHEST))


def _stage_e0_body(p0, p1, w, o):
    deg = jnp.maximum(p0[:, 1:2] + p1[:, 1:2], 1.0)
    a = (p0[:, 0:1] + p1[:, 0:1]) / deg
    o[...] = _lrelu(a * w[0:1, :])


def _tc_stage(parts, deg_parts, w, res=None):
    n_pad = parts.shape[1]
    d = parts.shape[2]
    bn = min(n_pad, 2048)
    grid = -(-n_pad // bn)
    row = lambda i: (i, 0)
    p0, p1 = parts[0], parts[1]
    if deg_parts is None:
        f = w.shape[1]
        wp = jnp.pad(w, ((0, 8 - w.shape[0]), (0, 0)))
        return pl.pallas_call(
            _stage_e0_body,
            grid=(grid,),
            in_specs=[pl.BlockSpec((bn, d), row), pl.BlockSpec((bn, d), row),
                      pl.BlockSpec((8, f), lambda i: (0, 0))],
            out_specs=pl.BlockSpec((bn, f), row),
            out_shape=jax.ShapeDtypeStruct((n_pad, f), jnp.float32),
        )(p0, p1, wp)
    f = w.shape[1]
    g0, g1 = deg_parts[0], deg_parts[1]
    specs = [pl.BlockSpec((bn, d), row), pl.BlockSpec((bn, d), row),
             pl.BlockSpec((bn, 16), row), pl.BlockSpec((bn, 16), row),
             pl.BlockSpec((d, f), lambda i: (0, 0))]
    args = [p0, p1, g0, g1, w]
    body = _stage_body
    if res is not None:
        specs.append(pl.BlockSpec((bn, f), row))
        args.append(res)
        body = _stage_body_res
    return pl.pallas_call(
        body,
        grid=(grid,),
        in_specs=specs,
        out_specs=pl.BlockSpec((bn, f), row),
        out_shape=jax.ShapeDtypeStruct((n_pad, f), jnp.float32),
    )(*args)


def _final_body(x, w, b, o):
    o[...] = lax.dot_general(w[...], x[...], (((0,), (1,)), ((), ())),
                             preferred_element_type=jnp.float32,
                             precision=lax.Precision.HIGHEST) + b[:, 0:1]


def _final(x, w, b):
    n_pad = x.shape[0]
    d = x.shape[1]
    f = w.shape[1]
    b2 = jnp.tile(b[:, None], (1, 8))
    return pl.pallas_call(
        _final_body,
        in_specs=[pl.BlockSpec((n_pad, d), lambda: (0, 0)),
                  pl.BlockSpec((d, f), lambda: (0, 0)),
                  pl.BlockSpec((f, 8), lambda: (0, 0))],
        out_specs=pl.BlockSpec((f, n_pad), lambda: (0, 0)),
        out_shape=jax.ShapeDtypeStruct((f, n_pad), jnp.float32),
    )(x, w, b2)


def kernel(features, edge_index0, pool_idx1, edge_index1, pool_idx2,
           edge_index2, pool_idx3, edge_index3, W_e0, W_e1, W_e2, W_l1a,
           W_l1b, W_l1c, W_l2a, W_l2b, W_l2c, W_x1, W_x2, W_out, b_out):
    np0 = _rup(_N0 + 1, 128)
    np1 = _rup(_N1 + 1, 128)
    np2 = _rup(_N2 + 1, 128)
    np3 = _rup(_N3 + 1, 128)
    s0, d0 = edge_index0[0], edge_index0[1]
    s1, d1 = edge_index1[0], edge_index1[1]
    s2, d2 = edge_index2[0], edge_index2[1]
    s3, d3 = edge_index3[0], edge_index3[1]

    x0aug = jnp.concatenate(
        [features, jnp.ones((_N0, 1), jnp.float32),
         jnp.zeros((_N0, 14), jnp.float32)], axis=1)
    x0aug = jnp.pad(x0aug, ((0, np0 - _N0), (0, 0)))
    agg0 = _aggregate(x0aug, s0, d0, np0)
    deg0 = agg0[:, :, 1:2]
    deg0 = jnp.pad(deg0, ((0, 0), (0, 0), (0, 15)))
    x = _tc_stage(agg0, None, W_e0)
    x = _tc_stage(_aggregate(x, s0, d0, np0), deg0, W_e1)
    x = _tc_stage(_aggregate(x, s0, d0, np0), deg0, W_e2, res=x)
    skip0 = x[:_N0]

    x = _pool(x, pool_idx1, np1)
    degp1 = _aggregate(jnp.ones((np1, 16), jnp.float32), s1, d1, np1)
    x = _tc_stage(_aggregate(x, s1, d1, np1), degp1, W_l1a)
    x = _tc_stage(_aggregate(x, s1, d1, np1), degp1, W_l1b, res=x)
    x = _tc_stage(_aggregate(x, s1, d1, np1), degp1, W_l1c, res=x)
    skip1 = x[:_N1]

    x = _pool(x, pool_idx2, np2)
    degp2 = _aggregate(jnp.ones((np2, 16), jnp.float32), s2, d2, np2)
    x = _tc_stage(_aggregate(x, s2, d2, np2), degp2, W_l2a)
    x = _tc_stage(_aggregate(x, s2, d2, np2), degp2, W_l2b, res=x)
    x = _tc_stage(_aggregate(x, s2, d2, np2), degp2, W_l2c, res=x)
    skip2 = x[:_N2]

    x = _pool(x, pool_idx3, np3)
    degp3 = _aggregate(jnp.ones((np3, 16), jnp.float32), s3, d3, np3)
    x = _tc_stage(_aggregate(x, s3, d3, np3), degp3, W_x1)
    x = _tc_stage(_aggregate(x, s3, d3, np3), degp3, W_x2, res=x)
    out_t = _final(x, W_out, b_out)
    final = out_t[:, :_N3][None, :, :]
    return (final, skip0, skip1, skip2)
